# Initial kernel scaffold; baseline (speedup 1.0000x reference)
import functools

import jax
import jax.numpy as jnp
from jax import lax
from jax.experimental import pallas as pl
from jax.experimental.pallas import tpu as pltpu
from jax.experimental.pallas import tpu_sc as plsc

VOCAB = 100000
EMB = 128
NCLASS = 100
B = 4096
L = 200

NCORES = 2
NSUB = 16
NW = NCORES * NSUB
BPW = B // NW
CH = 16
NCHUNK = BPW // CH
LANES = 16
KREG = EMB // LANES


def _sc_mean(x, emb_table):
    mesh = plsc.VectorSubcoreMesh(
        core_axis_name="c", subcore_axis_name="s",
        num_cores=NCORES, num_subcores=NSUB)

    @functools.partial(
        pl.kernel,
        mesh=mesh,
        out_type=jax.ShapeDtypeStruct((B, EMB), jnp.float32),
        scratch_types=[
            pltpu.VMEM((CH, L), jnp.int32),
            pltpu.VMEM((L, EMB), jnp.float32),
            pltpu.VMEM((CH, EMB), jnp.float32),
            pltpu.SemaphoreType.DMA,
        ],
    )
    def k(x_hbm, tab_hbm, mean_hbm, idx_v, rows_v, mst_v, sem):
        wid = lax.axis_index("s") * NCORES + lax.axis_index("c")
        base = wid * BPW

        def chunk_body(g, carry):
            row0 = base + g * CH
            pltpu.sync_copy(x_hbm.at[pl.ds(row0, CH)], idx_v)

            def seq_body(s, carry2):
                cp1 = pltpu.async_copy(
                    tab_hbm.at[idx_v.at[s, pl.ds(0, 128)]],
                    rows_v.at[pl.ds(0, 128)], sem)
                cp2 = pltpu.async_copy(
                    tab_hbm.at[idx_v.at[s, pl.ds(128, 72)]],
                    rows_v.at[pl.ds(128, 72)], sem)
                cp1.wait()
                cp2.wait()

                cnt = jnp.zeros((LANES,), jnp.int32)
                for j in range(12):
                    v = idx_v[s, j * 16:(j + 1) * 16]
                    cnt = cnt + jnp.where(v != 0, 1, 0).astype(jnp.int32)
                lane = lax.iota(jnp.int32, 16)
                vtail = idx_v[s, 184:200]
                cnt = cnt + jnp.where(
                    (vtail != 0) & (lane >= 8), 1, 0).astype(jnp.int32)
                c = jnp.sum(cnt)
                inv = 1.0 / jnp.maximum(c.astype(jnp.float32), 1.0)

                def acc_body(j, acc):
                    j0 = j * 4
                    for u in range(4):
                        acc = tuple(
                            acc[kk] + rows_v[j0 + u, kk * 16:(kk + 1) * 16]
                            for kk in range(KREG))
                    return acc

                acc0 = tuple(jnp.zeros((LANES,), jnp.float32)
                             for _ in range(KREG))
                acc = lax.fori_loop(0, L // 4, acc_body, acc0)

                for kk in range(KREG):
                    mst_v[s, kk * 16:(kk + 1) * 16] = acc[kk] * inv
                return carry2

            lax.fori_loop(0, CH, seq_body, 0)
            pltpu.sync_copy(mst_v, mean_hbm.at[pl.ds(row0, CH)])
            return carry

        lax.fori_loop(0, NCHUNK, chunk_body, 0)

    return k(x, emb_table)


def _mlp(mean, W1, b1, W2, b2):
    def body(m_ref, w1_ref, b1_ref, w2_ref, b2_ref, o_ref):
        m = m_ref[...]
        h = lax.dot_general(m, w1_ref[...], (((1,), (1,)), ((), ())),
                            preferred_element_type=jnp.float32)
        h = jnp.maximum(h + b1_ref[...], 0.0)
        o = lax.dot_general(h, w2_ref[...], (((1,), (1,)), ((), ())),
                            preferred_element_type=jnp.float32)
        o_ref[...] = o + b2_ref[...]

    nblk = 4
    return pl.pallas_call(
        body,
        out_shape=jax.ShapeDtypeStruct((B, NCLASS), jnp.float32),
        grid=(nblk,),
        in_specs=[
            pl.BlockSpec((B // nblk, EMB), lambda i: (i, 0)),
            pl.BlockSpec((128, EMB), lambda i: (0, 0)),
            pl.BlockSpec((1, 128), lambda i: (0, 0)),
            pl.BlockSpec((NCLASS, 128), lambda i: (0, 0)),
            pl.BlockSpec((1, NCLASS), lambda i: (0, 0)),
        ],
        out_specs=pl.BlockSpec((B // nblk, NCLASS), lambda i: (i, 0)),
    )(mean, W1, b1.reshape(1, 128), W2, b2.reshape(1, NCLASS))


def kernel(x, lengths, emb_table, W1, b1, W2, b2):
    mean = _sc_mean(x, emb_table)
    return _mlp(mean, W1, b1, W2, b2)


# speedup vs baseline: 8.8480x; 8.8480x over previous
import functools

import jax
import jax.numpy as jnp
from jax import lax
from jax.experimental import pallas as pl
from jax.experimental.pallas import tpu as pltpu
from jax.experimental.pallas import tpu_sc as plsc

VOCAB = 100000
EMB = 128
NCLASS = 100
B = 4096
L = 200

NCORES = 2
NSUB = 16
NW = NCORES * NSUB
BPW = B // NW
CH = 16
NCHUNK = BPW // CH
LANES = 16
KREG = EMB // LANES


def _sc_mean(x, emb_table):
    mesh = plsc.VectorSubcoreMesh(
        core_axis_name="c", subcore_axis_name="s",
        num_cores=NCORES, num_subcores=NSUB)

    @functools.partial(
        pl.kernel,
        mesh=mesh,
        out_type=jax.ShapeDtypeStruct((B, EMB), jnp.float32),
        scratch_types=[
            pltpu.VMEM((CH, L), jnp.int32),
            pltpu.VMEM((L, EMB), jnp.float32),
            pltpu.VMEM((CH, EMB), jnp.float32),
            pltpu.SemaphoreType.DMA,
        ],
        compiler_params=pltpu.CompilerParams(needs_layout_passes=False),
    )
    def k(x_hbm, tab_hbm, mean_hbm, idx_v, rows_v, mst_v, sem):
        wid = lax.axis_index("s") * NCORES + lax.axis_index("c")
        base = wid * BPW

        def chunk_body(g, carry):
            row0 = base + g * CH
            pltpu.sync_copy(x_hbm.at[pl.ds(row0, CH)], idx_v)

            def seq_body(s, carry2):
                cp1 = pltpu.async_copy(
                    tab_hbm.at[idx_v.at[s, pl.ds(0, 128)]],
                    rows_v.at[pl.ds(0, 128)], sem)
                cp2 = pltpu.async_copy(
                    tab_hbm.at[idx_v.at[s, pl.ds(128, 72)]],
                    rows_v.at[pl.ds(128, 72)], sem)
                cp1.wait()
                cp2.wait()

                cnt = jnp.zeros((LANES,), jnp.int32)
                for j in range(12):
                    v = idx_v[s, j * 16:(j + 1) * 16]
                    cnt = cnt + plsc.all_reduce_population_count(v != 0)
                lane = lax.iota(jnp.int32, 16)
                vtail = idx_v[s, 184:200]
                cnt = cnt + plsc.all_reduce_population_count(
                    (vtail != 0) & (lane >= 8))
                inv = 1.0 / jnp.maximum(cnt.astype(jnp.float32), 1.0)

                def acc_body(j, acc):
                    j0 = j * 4
                    for u in range(4):
                        acc = tuple(
                            acc[kk] + rows_v[j0 + u, kk * 16:(kk + 1) * 16]
                            for kk in range(KREG))
                    return acc

                acc0 = tuple(jnp.zeros((LANES,), jnp.float32)
                             for _ in range(KREG))
                acc = lax.fori_loop(0, L // 4, acc_body, acc0)

                for kk in range(KREG):
                    mst_v[s, kk * 16:(kk + 1) * 16] = acc[kk] * inv
                return carry2

            lax.fori_loop(0, CH, seq_body, 0)
            pltpu.sync_copy(mst_v, mean_hbm.at[pl.ds(row0, CH)])
            return carry

        lax.fori_loop(0, NCHUNK, chunk_body, 0)

    return k(x, emb_table)


def _mlp(mean, W1, b1, W2, b2):
    def body(m_ref, w1_ref, b1_ref, w2_ref, b2_ref, o_ref):
        m = m_ref[...]
        h = lax.dot_general(m, w1_ref[...], (((1,), (1,)), ((), ())),
                            preferred_element_type=jnp.float32)
        h = jnp.maximum(h + b1_ref[...], 0.0)
        o = lax.dot_general(h, w2_ref[...], (((1,), (1,)), ((), ())),
                            preferred_element_type=jnp.float32)
        o_ref[...] = o + b2_ref[...]

    nblk = 4
    return pl.pallas_call(
        body,
        out_shape=jax.ShapeDtypeStruct((B, NCLASS), jnp.float32),
        grid=(nblk,),
        in_specs=[
            pl.BlockSpec((B // nblk, EMB), lambda i: (i, 0)),
            pl.BlockSpec((128, EMB), lambda i: (0, 0)),
            pl.BlockSpec((1, 128), lambda i: (0, 0)),
            pl.BlockSpec((NCLASS, 128), lambda i: (0, 0)),
            pl.BlockSpec((1, NCLASS), lambda i: (0, 0)),
        ],
        out_specs=pl.BlockSpec((B // nblk, NCLASS), lambda i: (i, 0)),
    )(mean, W1, b1.reshape(1, 128), W2, b2.reshape(1, NCLASS))


def kernel(x, lengths, emb_table, W1, b1, W2, b2):
    mean = _sc_mean(x, emb_table)
    return _mlp(mean, W1, b1, W2, b2)


# speedup vs baseline: 14.3557x; 1.6225x over previous
import functools

import jax
import jax.numpy as jnp
from jax import lax
from jax.experimental import pallas as pl
from jax.experimental.pallas import tpu as pltpu
from jax.experimental.pallas import tpu_sc as plsc

VOCAB = 100000
EMB = 128
NCLASS = 100
B = 4096
L = 200

NCORES = 2
NSUB = 16
NW = NCORES * NSUB
BPW = B // NW
CH = 16
NCHUNK = BPW // CH
LANES = 16
KREG = EMB // LANES


def _sc_mean(x, emb_table):
    mesh = plsc.VectorSubcoreMesh(
        core_axis_name="c", subcore_axis_name="s",
        num_cores=NCORES, num_subcores=NSUB)

    @functools.partial(
        pl.kernel,
        mesh=mesh,
        out_type=jax.ShapeDtypeStruct((B, EMB), jnp.float32),
        scratch_types=[
            pltpu.VMEM((CH, L), jnp.int32),
            pltpu.VMEM((2, L, EMB), jnp.float32),
            pltpu.VMEM((CH, EMB), jnp.float32),
            pltpu.SemaphoreType.DMA,
            pltpu.SemaphoreType.DMA,
        ],
        compiler_params=pltpu.CompilerParams(needs_layout_passes=False),
    )
    def k(x_hbm, tab_hbm, mean_hbm, idx_v, rows_v, mst_v, sem0, sem1):
        wid = lax.axis_index("s") * NCORES + lax.axis_index("c")
        base = wid * BPW

        def issue(s, buf, sem):
            pltpu.async_copy(
                tab_hbm.at[idx_v.at[s, pl.ds(0, 128)]],
                rows_v.at[buf, pl.ds(0, 128)], sem)
            pltpu.async_copy(
                tab_hbm.at[idx_v.at[s, pl.ds(128, 72)]],
                rows_v.at[buf, pl.ds(128, 72)], sem)

        def wait(buf, sem):
            pltpu.make_async_copy(
                tab_hbm.at[idx_v.at[0, pl.ds(0, 128)]],
                rows_v.at[buf, pl.ds(0, 128)], sem).wait()
            pltpu.make_async_copy(
                tab_hbm.at[idx_v.at[0, pl.ds(128, 72)]],
                rows_v.at[buf, pl.ds(128, 72)], sem).wait()

        def consume(s, buf):
            cnt = jnp.zeros((LANES,), jnp.int32)
            for j in range(12):
                v = idx_v[s, j * 16:(j + 1) * 16]
                cnt = cnt + plsc.all_reduce_population_count(v != 0)
            lane = lax.iota(jnp.int32, 16)
            vtail = idx_v[s, 184:200]
            cnt = cnt + plsc.all_reduce_population_count(
                (vtail != 0) & (lane >= 8))
            inv = 1.0 / jnp.maximum(cnt.astype(jnp.float32), 1.0)

            def acc_body(j, acc):
                j0 = j * 8
                for u in range(8):
                    acc = tuple(
                        acc[kk] + rows_v[buf, j0 + u, kk * 16:(kk + 1) * 16]
                        for kk in range(KREG))
                return acc

            acc0 = tuple(jnp.zeros((LANES,), jnp.float32)
                         for _ in range(KREG))
            acc = lax.fori_loop(0, L // 8, acc_body, acc0)

            for kk in range(KREG):
                mst_v[s, kk * 16:(kk + 1) * 16] = acc[kk] * inv

        def chunk_body(g, carry):
            row0 = base + g * CH
            pltpu.sync_copy(x_hbm.at[pl.ds(row0, CH)], idx_v)
            issue(0, 0, sem0)

            def pair_body(t, carry2):
                s0 = t * 2
                issue(s0 + 1, 1, sem1)
                wait(0, sem0)
                consume(s0, 0)

                @pl.when(t != CH // 2 - 1)
                def _():
                    issue(s0 + 2, 0, sem0)

                wait(1, sem1)
                consume(s0 + 1, 1)
                return carry2

            lax.fori_loop(0, CH // 2, pair_body, 0)
            pltpu.sync_copy(mst_v, mean_hbm.at[pl.ds(row0, CH)])
            return carry

        lax.fori_loop(0, NCHUNK, chunk_body, 0)

    return k(x, emb_table)


def _mlp(mean, W1, b1, W2, b2):
    def body(m_ref, w1_ref, b1_ref, w2_ref, b2_ref, o_ref):
        m = m_ref[...]
        h = lax.dot_general(m, w1_ref[...], (((1,), (1,)), ((), ())),
                            preferred_element_type=jnp.float32)
        h = jnp.maximum(h + b1_ref[...], 0.0)
        o = lax.dot_general(h, w2_ref[...], (((1,), (1,)), ((), ())),
                            preferred_element_type=jnp.float32)
        o_ref[...] = o + b2_ref[...]

    nblk = 4
    return pl.pallas_call(
        body,
        out_shape=jax.ShapeDtypeStruct((B, NCLASS), jnp.float32),
        grid=(nblk,),
        in_specs=[
            pl.BlockSpec((B // nblk, EMB), lambda i: (i, 0)),
            pl.BlockSpec((128, EMB), lambda i: (0, 0)),
            pl.BlockSpec((1, 128), lambda i: (0, 0)),
            pl.BlockSpec((NCLASS, 128), lambda i: (0, 0)),
            pl.BlockSpec((1, NCLASS), lambda i: (0, 0)),
        ],
        out_specs=pl.BlockSpec((B // nblk, NCLASS), lambda i: (i, 0)),
    )(mean, W1, b1.reshape(1, 128), W2, b2.reshape(1, NCLASS))


def kernel(x, lengths, emb_table, W1, b1, W2, b2):
    mean = _sc_mean(x, emb_table)
    return _mlp(mean, W1, b1, W2, b2)
